# Initial kernel scaffold; baseline (speedup 1.0000x reference)
import jax
import jax.numpy as jnp
from jax.experimental import pallas as pl


def kernel(x, side_to_move, emb, W1, b1, W2, b2, W3, b3):
    raise NotImplementedError("write your pallas kernel here")


# speedup vs baseline: 24.0608x; 24.0608x over previous
import functools

import jax
import jax.numpy as jnp
from jax import lax
from jax.experimental import pallas as pl
from jax.experimental.pallas import tpu as pltpu
from jax.experimental.pallas import tpu_sc as plsc

VOCAB = 832
EMB_DIM = 1024
D1 = 512
W_CNT = 848
NC = 2
NS = 16
NW = NC * NS
LANES = 16


def _sc_counts(x_flat, side):
    B = side.shape[0]
    b_per_w = B // NW
    CH = LANES
    n_ch = b_per_w // CH

    mesh = plsc.VectorSubcoreMesh(core_axis_name="c", subcore_axis_name="s")

    @functools.partial(
        pl.kernel,
        out_type=jax.ShapeDtypeStruct((B * W_CNT,), jnp.float32),
        mesh=mesh,
        compiler_params=pltpu.CompilerParams(needs_layout_passes=False),
        scratch_types=[
            pltpu.VMEM((CH * 64,), jnp.int32),
            pltpu.VMEM((CH,), jnp.float32),
            pltpu.VMEM((CH * W_CNT,), jnp.float32),
        ],
    )
    def k(x_hbm, side_hbm, out_hbm, idx_v, side_v, cnt_v):
        wid = lax.axis_index("s") * NC + lax.axis_index("c")
        row0w = wid * b_per_w
        lane = lax.iota(jnp.int32, 16)
        rowbase = lane * W_CNT
        colbase = lane * 64
        ones = jnp.ones((16,), jnp.float32)
        zeros = jnp.zeros((16,), jnp.float32)

        for i in range(CH * W_CNT // 16):
            cnt_v[pl.ds(i * 16, 16)] = zeros

        def chunk_body(ch, carry):
            row0 = row0w + ch * CH
            pltpu.sync_copy(x_hbm.at[pl.ds(row0 * 64, CH * 64)], idx_v)
            pltpu.sync_copy(side_hbm.at[pl.ds(row0, CH)], side_v)
            for sq in range(64):
                iv = plsc.load_gather(idx_v, [colbase + sq])
                plsc.addupdate_scatter(cnt_v, [rowbase + iv], ones)
            plsc.store_scatter(cnt_v, [rowbase + VOCAB], side_v[...])
            pltpu.sync_copy(cnt_v, out_hbm.at[pl.ds(row0 * W_CNT, CH * W_CNT)])
            for sq in range(64):
                iv = plsc.load_gather(idx_v, [colbase + sq])
                plsc.store_scatter(cnt_v, [rowbase + iv], zeros)
            return carry

        lax.fori_loop(0, n_ch, chunk_body, 0)

    return k(x_flat, side)


def _tc_fold(emb, w1a, w1s):
    def body(emb_ref, w1a_ref, w1s_ref, out_ref):
        m = lax.dot_general(
            emb_ref[...], w1a_ref[...], (((1,), (1,)), ((), ())),
            preferred_element_type=jnp.float32)
        pad = jnp.zeros((W_CNT - VOCAB - 1, D1), jnp.float32)
        out_ref[...] = jnp.concatenate([m, w1s_ref[...], pad], axis=0)

    return pl.pallas_call(
        body,
        out_shape=jax.ShapeDtypeStruct((W_CNT, D1), jnp.float32),
    )(emb, w1a, w1s)


def _tc_mlp(cnts, M, b1, W2, b2, W3, b3):
    B = cnts.shape[0]
    BT = 512
    nb = B // BT

    def body(c_ref, m_ref, b1_ref, w2_ref, b2_ref, w3_ref, b3_ref, o_ref):
        g = jnp.dot(c_ref[...], m_ref[...], preferred_element_type=jnp.float32)
        h1 = jnp.maximum(g + b1_ref[...], 0.0)
        h2 = lax.dot_general(h1, w2_ref[...], (((1,), (1,)), ((), ())),
                             preferred_element_type=jnp.float32)
        h2 = jnp.maximum(h2 + b2_ref[...], 0.0)
        h3 = jnp.sum(h2 * w3_ref[...], axis=1, keepdims=True)
        o_ref[...] = h3 + b3_ref[0, 0]

    return pl.pallas_call(
        body,
        grid=(nb,),
        in_specs=[
            pl.BlockSpec((BT, W_CNT), lambda i: (i, 0)),
            pl.BlockSpec((W_CNT, D1), lambda i: (0, 0)),
            pl.BlockSpec((1, D1), lambda i: (0, 0)),
            pl.BlockSpec((256, D1), lambda i: (0, 0)),
            pl.BlockSpec((1, 256), lambda i: (0, 0)),
            pl.BlockSpec((1, 256), lambda i: (0, 0)),
            pl.BlockSpec(memory_space=pltpu.SMEM),
        ],
        out_specs=pl.BlockSpec((BT, 1), lambda i: (i, 0)),
        out_shape=jax.ShapeDtypeStruct((B, 1), jnp.float32),
    )(cnts, M, b1, W2, b2, W3, b3)


def kernel(x, side_to_move, emb, W1, b1, W2, b2, W3, b3):
    B = x.shape[0]
    x_flat = x.astype(jnp.int32).reshape(B * 64)
    w1a = W1[:, :EMB_DIM]
    w1s = W1[:, EMB_DIM].reshape(1, D1)
    cnts = _sc_counts(x_flat, side_to_move).reshape(B, W_CNT)
    M = _tc_fold(emb, w1a, w1s)
    return _tc_mlp(cnts, M, b1.reshape(1, D1), W2, b2.reshape(1, 256),
                   W3, b3.reshape(1, 1))
